# baseline (speedup 1.0000x reference)
import functools

import jax
import jax.numpy as jnp
from jax import lax
from jax.experimental import pallas as pl
from jax.experimental.pallas import tpu as pltpu
from jax.experimental.pallas import tpu_sc as plsc

N_NODES = 10000
D = 256
NC = 2
NS = 16
NW = NC * NS

NPAD = 10240
ROWS_PER_TILE = NPAD // NW
NHIST = 10368
CH = 128
SCAN = 3072
CHD = 64
BM = 512

def _sc_mesh():
    return plsc.VectorSubcoreMesh(
        core_axis_name="c", subcore_axis_name="s",
        num_cores=NC, num_subcores=NS)


def _hist_body(ept_a, dst_hbm, out_hbm, hist, dstbuf):
    cid = lax.axis_index("c")
    sid = lax.axis_index("s")
    wid = cid * NS + sid
    pltpu.sync_copy(dst_hbm.at[pl.ds(wid * ept_a, ept_a)], dstbuf)
    zeros = jnp.zeros((16,), jnp.float32)

    def zbody(i, c):
        hist[pl.ds(i * 16, 16)] = zeros
        return c
    lax.fori_loop(0, NHIST // 16, zbody, 0)

    ones = jnp.ones((16,), jnp.float32)

    def body(i, c):
        dv = dstbuf[pl.ds(i * 16, 16)]
        plsc.addupdate_scatter(hist, [dv], ones)
        return c
    lax.fori_loop(0, ept_a // 16, body, 0)
    pltpu.sync_copy(hist, out_hbm.at[wid])


def _propagate_body(epad, z_hbm, src_hbm, dst_hbm, acc_hbm,
                    acc, gbuf, srcc, dstc, wl_src, wl_dst, gsem):
    cid = lax.axis_index("c")
    sid = lax.axis_index("s")
    wid = cid * NS + sid
    row_lo = wid * ROWS_PER_TILE

    pltpu.sync_copy(z_hbm.at[pl.ds(row_lo, ROWS_PER_TILE)],
                    acc.at[pl.ds(0, ROWS_PER_TILE)])

    zi = jnp.zeros((16,), jnp.int32)

    def zb(i, c):
        wl_src[pl.ds(i * 16, 16)] = zi
        wl_dst[pl.ds(i * 16, 16)] = zi
        return c
    lax.fori_loop(0, (SCAN + CHD) // 16, zb, 0)

    def accumulate_batch(b, slot):
        for rb in range(CHD // 16):
            locv = wl_dst[pl.ds(b * CHD + rb * 16, 16)]
            for l in range(16):
                loc = locv[l]
                r = rb * 16 + l
                for cc in range(D // 16):
                    plsc.addupdate(acc.at[loc, pl.ds(cc * 16, 16)],
                                   gbuf[slot, r, pl.ds(cc * 16, 16)])

    def drain_batches(nb, wcount):
        @pl.when(nb > 0)
        def _():
            pltpu.async_copy(
                z_hbm.at[wl_src.at[pl.ds(0, CHD)]], gbuf.at[0], gsem)

        def dbody(b, c):
            slot = lax.rem(b, 2)
            pltpu.make_async_copy(
                z_hbm.at[wl_src.at[pl.ds(0, CHD)]], gbuf.at[slot], gsem
            ).wait()

            @pl.when(b + 1 < nb)
            def _():
                pltpu.async_copy(
                    z_hbm.at[wl_src.at[pl.ds((b + 1) * CHD, CHD)]],
                    gbuf.at[1 - slot], gsem)
            accumulate_batch(b, slot)
            return c
        lax.fori_loop(0, nb, dbody, 0)
        base = nb * CHD

        def mv(k, c):
            sv = wl_src[pl.ds(base + k * 16, 16)]
            dv = wl_dst[pl.ds(base + k * 16, 16)]
            wl_src[pl.ds(k * 16, 16)] = sv
            wl_dst[pl.ds(k * 16, 16)] = dv
            return c
        lax.fori_loop(0, CHD // 16, mv, 0)
        return wcount - nb * CHD

    def sbody(sc_i, wcount):
        off = sc_i * SCAN
        pltpu.sync_copy(src_hbm.at[pl.ds(off, SCAN)], srcc)
        pltpu.sync_copy(dst_hbm.at[pl.ds(off, SCAN)], dstc)

        def vbody(v, wc):
            d = dstc[pl.ds(v * 16, 16)]
            s = srcc[pl.ds(v * 16, 16)]
            loc = d - row_lo
            mask = loc.astype(jnp.uint32) < jnp.uint32(ROWS_PER_TILE)
            plsc.store_compressed(wl_src.at[pl.ds(wc, 16)], s, mask=mask)
            plsc.store_compressed(wl_dst.at[pl.ds(wc, 16)], loc, mask=mask)
            cnt = plsc.all_reduce_population_count(mask)
            return wc + cnt[0]
        wcount = lax.fori_loop(0, SCAN // 16, vbody, wcount)
        return drain_batches(wcount // CHD, wcount)
    wcount = lax.fori_loop(0, epad // SCAN, sbody, 0)

    @pl.when(wcount > 0)
    def _():
        dummy = jnp.full((16,), ROWS_PER_TILE, jnp.int32)
        for k in range(CHD // 16):
            wl_dst[pl.ds(wcount + k * 16, 16)] = dummy
        pltpu.sync_copy(z_hbm.at[wl_src.at[pl.ds(0, CHD)]], gbuf.at[0])
        accumulate_batch(0, 0)

    pltpu.sync_copy(acc.at[pl.ds(0, ROWS_PER_TILE)],
                    acc_hbm.at[pl.ds(row_lo, ROWS_PER_TILE)])


def _hist_call(dst_p, epad):
    ept_a = epad // NW
    return pl.kernel(
        functools.partial(_hist_body, ept_a),
        out_type=jax.ShapeDtypeStruct((NW, NHIST), jnp.float32),
        mesh=_sc_mesh(),
        compiler_params=pltpu.CompilerParams(needs_layout_passes=False),
        scratch_types=[
            pltpu.VMEM((NHIST,), jnp.float32),
            pltpu.VMEM((ept_a,), jnp.int32),
        ],
    )(dst_p)


def _propagate_call(z, src_p, dst_p, epad):
    return pl.kernel(
        functools.partial(_propagate_body, epad),
        out_type=jax.ShapeDtypeStruct((NPAD, D), jnp.float32),
        mesh=_sc_mesh(),
        compiler_params=pltpu.CompilerParams(needs_layout_passes=False),
        scratch_types=[
            pltpu.VMEM((ROWS_PER_TILE + 1, D), jnp.float32),
            pltpu.VMEM((2, CHD, D), jnp.float32),
            pltpu.VMEM((SCAN,), jnp.int32),
            pltpu.VMEM((SCAN,), jnp.int32),
            pltpu.VMEM((SCAN + CHD + 16,), jnp.int32),
            pltpu.VMEM((SCAN + CHD + 16,), jnp.int32),
            pltpu.SemaphoreType.DMA,
        ],
    )(z, src_p, dst_p)


def _dinv_body(hist_ref, o_ref):
    deg = jnp.sum(hist_ref[...], axis=0, keepdims=True) + 1.0
    o_ref[...] = lax.rsqrt(deg)


def _mm_scale_body(x_ref, w_ref, dinv_ref, o_ref):
    o_ref[...] = jnp.dot(x_ref[...], w_ref[...],
                         preferred_element_type=jnp.float32) * dinv_ref[...]


def _layer2_body(acc_ref, dinv_ref, b_ref, w_ref, o_ref):
    h = jnp.maximum(acc_ref[...] * dinv_ref[...] + b_ref[...], 0.0)
    o_ref[...] = jnp.dot(h, w_ref[...],
                         preferred_element_type=jnp.float32) * dinv_ref[...]


def _final_body(acc_ref, dinv_ref, b_ref, o_ref):
    o_ref[...] = acc_ref[...] * dinv_ref[...] + b_ref[...]


def kernel(x, edge_index, W1, b1, W2, b2):
    n, d = x.shape
    e = edge_index.shape[1]
    assert n == N_NODES and d == D
    epad = -(-e // SCAN) * SCAN

    ei = edge_index.astype(jnp.int32)
    src_p = jnp.concatenate([ei[0], jnp.zeros((epad - e,), jnp.int32)])
    dst_p = jnp.concatenate(
        [ei[1], jnp.full((epad - e,), NPAD, jnp.int32)])
    x_p = jnp.concatenate(
        [x, jnp.zeros((NPAD - n, d), jnp.float32)], axis=0)
    b1r = b1.reshape(1, d)
    b2r = b2.reshape(1, d)

    hist = _hist_call(dst_p, epad)

    dinv_row = pl.pallas_call(
        _dinv_body,
        in_specs=[pl.BlockSpec((NW, NHIST), lambda: (0, 0))],
        out_specs=pl.BlockSpec((1, NHIST), lambda: (0, 0)),
        out_shape=jax.ShapeDtypeStruct((1, NHIST), jnp.float32),
    )(hist)
    dinv_col = dinv_row[0, :NPAD].reshape(NPAD, 1)

    grid = (NPAD // BM,)
    z1 = pl.pallas_call(
        _mm_scale_body,
        grid=grid,
        in_specs=[pl.BlockSpec((BM, d), lambda i: (i, 0)),
                  pl.BlockSpec((d, d), lambda i: (0, 0)),
                  pl.BlockSpec((BM, 1), lambda i: (i, 0))],
        out_specs=pl.BlockSpec((BM, d), lambda i: (i, 0)),
        out_shape=jax.ShapeDtypeStruct((NPAD, d), jnp.float32),
    )(x_p, W1, dinv_col)

    acc1 = _propagate_call(z1, src_p, dst_p, epad)

    z2 = pl.pallas_call(
        _layer2_body,
        grid=grid,
        in_specs=[pl.BlockSpec((BM, d), lambda i: (i, 0)),
                  pl.BlockSpec((BM, 1), lambda i: (i, 0)),
                  pl.BlockSpec((1, d), lambda i: (0, 0)),
                  pl.BlockSpec((d, d), lambda i: (0, 0))],
        out_specs=pl.BlockSpec((BM, d), lambda i: (i, 0)),
        out_shape=jax.ShapeDtypeStruct((NPAD, d), jnp.float32),
    )(acc1, dinv_col, b1r, W2)

    acc2 = _propagate_call(z2, src_p, dst_p, epad)

    out = pl.pallas_call(
        _final_body,
        grid=grid,
        in_specs=[pl.BlockSpec((BM, d), lambda i: (i, 0)),
                  pl.BlockSpec((BM, 1), lambda i: (i, 0)),
                  pl.BlockSpec((1, d), lambda i: (0, 0))],
        out_specs=pl.BlockSpec((BM, d), lambda i: (i, 0)),
        out_shape=jax.ShapeDtypeStruct((NPAD, d), jnp.float32),
    )(acc2, dinv_col, b2r)
    return out[:n]


# speedup vs baseline: 1.4539x; 1.4539x over previous
import functools

import jax
import jax.numpy as jnp
from jax import lax
from jax.experimental import pallas as pl
from jax.experimental.pallas import tpu as pltpu
from jax.experimental.pallas import tpu_sc as plsc

N_NODES = 10000
D = 256
NC = 2
NS = 16
NW = NC * NS

NPAD = 10240
RPT = NPAD // NW
SCAN = 8192
CHD = 64
WB = 2048
BM = 512
DUMMY_PACKED = RPT


def _sc_mesh():
    return plsc.VectorSubcoreMesh(
        core_axis_name="c", subcore_axis_name="s",
        num_cores=NC, num_subcores=NS)


def _cp():
    return pltpu.CompilerParams(needs_layout_passes=False)


def _build_body(epad, wl_cap, src_hbm, dst_hbm, wl_hbm, cnt_hbm, hist_hbm,
                srcc, dstc, wlb, histloc, cntbuf):
    cid = lax.axis_index("c")
    sid = lax.axis_index("s")
    wid = cid * NS + sid
    row_lo = wid * RPT

    zeros = jnp.zeros((16,), jnp.float32)

    def zb(i, c):
        histloc[pl.ds(i * 16, 16)] = zeros
        return c
    lax.fori_loop(0, RPT // 16, zb, 0)

    ones = jnp.ones((16,), jnp.float32)
    dummy16 = jnp.full((16,), DUMMY_PACKED, jnp.int32)
    iota16 = lax.iota(jnp.int32, 16)

    def sbody(sc_i, carry):
        g_off = carry
        off = sc_i * SCAN
        pltpu.sync_copy(src_hbm.at[pl.ds(off, SCAN)], srcc)
        pltpu.sync_copy(dst_hbm.at[pl.ds(off, SCAN)], dstc)

        def vbody(v, wc):
            d = dstc[pl.ds(v * 16, 16)]
            s = srcc[pl.ds(v * 16, 16)]
            loc = d - row_lo
            mask = loc.astype(jnp.uint32) < jnp.uint32(RPT)
            packed = (s << 9) | loc
            plsc.store_compressed(wlb.at[pl.ds(wc, 16)], packed, mask=mask)
            plsc.addupdate_scatter(histloc, [loc], ones, mask=mask)
            cnt = plsc.all_reduce_population_count(mask)
            return wc + cnt[0]
        wc = lax.fori_loop(0, SCAN // 16, vbody, 0)
        plsc.store_compressed(wlb.at[pl.ds(wc, 16)], dummy16,
                              mask=(iota16 >= 0))
        wc16 = ((wc + 15) // 16) * 16
        pltpu.sync_copy(
            wlb.at[pl.ds(0, SCAN)],
            wl_hbm.at[pl.ds(pl.multiple_of(wid * wl_cap + g_off, 16), SCAN)])
        return g_off + wc16
    g_off = lax.fori_loop(0, epad // SCAN, sbody, 0)

    for k in range(CHD // 16):
        wlb[pl.ds(k * 16, 16)] = dummy16
    pltpu.sync_copy(
        wlb.at[pl.ds(0, CHD)],
        wl_hbm.at[pl.ds(pl.multiple_of(wid * wl_cap + g_off, 16), CHD)])
    total = ((g_off + CHD - 1) // CHD) * CHD

    cntbuf[...] = jnp.broadcast_to(total, (16,)).astype(jnp.int32)
    pltpu.sync_copy(cntbuf, cnt_hbm.at[pl.ds(wid * 16, 16)])
    pltpu.sync_copy(histloc.at[pl.ds(0, RPT)],
                    hist_hbm.at[pl.ds(wid * RPT, RPT)])


def _propagate_body(wl_cap, z_hbm, wl_hbm, cnt_hbm, acc_hbm,
                    acc, gbuf, pbuf, srcb, locb, cntbuf, gsem):
    cid = lax.axis_index("c")
    sid = lax.axis_index("s")
    wid = cid * NS + sid
    row_lo = wid * RPT

    pltpu.sync_copy(z_hbm.at[pl.ds(row_lo, RPT)], acc.at[pl.ds(0, RPT)])

    pltpu.sync_copy(cnt_hbm.at[pl.ds(wid * 16, 16)], cntbuf)
    total = cntbuf[...][0]

    iota16 = lax.iota(jnp.int32, 16)
    splats = [jnp.full((16,), kl, jnp.int32) for kl in range(16)]
    colvs = [iota16 + kc * 16 for kc in range(D // 16)]

    def accumulate_batch(boff, slot):
        gslot = gbuf.at[slot]
        for g in range(CHD // 16):
            locv = locb[pl.ds(boff + g * 16, 16)]
            for l in range(16):
                r = g * 16 + l
                rowsplat = locv[splats[l]]
                for kc in range(D // 16):
                    v = gslot[r, pl.ds(kc * 16, 16)]
                    plsc.addupdate_scatter(acc, [rowsplat, colvs[kc]], v)

    nblocks = (total + WB - 1) // WB

    def bbody(bb, c):
        pltpu.sync_copy(
            wl_hbm.at[pl.ds(pl.multiple_of(wid * wl_cap + bb * WB, 16), WB)],
            pbuf)

        def ub(j, c2):
            v = pbuf[pl.ds(j * 16, 16)]
            srcb[pl.ds(j * 16, 16)] = v >> 9
            locb[pl.ds(j * 16, 16)] = v & 511
            return c2
        lax.fori_loop(0, WB // 16, ub, 0)

        rem = total - bb * WB
        nbb = jnp.minimum(rem, WB) // CHD

        @pl.when(nbb > 0)
        def _():
            pltpu.async_copy(
                z_hbm.at[srcb.at[pl.ds(0, CHD)]], gbuf.at[0], gsem)

            def dbody(b, c3):
                slot = lax.rem(b, 2)
                pltpu.make_async_copy(
                    z_hbm.at[srcb.at[pl.ds(0, CHD)]], gbuf.at[slot], gsem
                ).wait()

                @pl.when(b + 1 < nbb)
                def _():
                    pltpu.async_copy(
                        z_hbm.at[srcb.at[pl.ds((b + 1) * CHD, CHD)]],
                        gbuf.at[1 - slot], gsem)
                accumulate_batch(b * CHD, slot)
                return c3
            lax.fori_loop(0, nbb, dbody, 0)
        return c
    lax.fori_loop(0, nblocks, bbody, 0)

    pltpu.sync_copy(acc.at[pl.ds(0, RPT)],
                    acc_hbm.at[pl.ds(row_lo, RPT)])


def _build_call(src_p, dst_p, epad):
    wl_cap = epad + SCAN + 512
    return pl.kernel(
        functools.partial(_build_body, epad, wl_cap),
        out_type=(jax.ShapeDtypeStruct((NW * wl_cap,), jnp.int32),
                  jax.ShapeDtypeStruct((NW * 16,), jnp.int32),
                  jax.ShapeDtypeStruct((NW * RPT,), jnp.float32)),
        mesh=_sc_mesh(),
        compiler_params=_cp(),
        scratch_types=[
            pltpu.VMEM((SCAN,), jnp.int32),
            pltpu.VMEM((SCAN,), jnp.int32),
            pltpu.VMEM((SCAN + 32,), jnp.int32),
            pltpu.VMEM((RPT,), jnp.float32),
            pltpu.VMEM((16,), jnp.int32),
        ],
    )(src_p, dst_p)


def _propagate_call(z, wl, cnt, wl_cap):
    return pl.kernel(
        functools.partial(_propagate_body, wl_cap),
        out_type=jax.ShapeDtypeStruct((NPAD, D), jnp.float32),
        mesh=_sc_mesh(),
        compiler_params=_cp(),
        scratch_types=[
            pltpu.VMEM((RPT + 1, D), jnp.float32),
            pltpu.VMEM((2, CHD, D), jnp.float32),
            pltpu.VMEM((WB,), jnp.int32),
            pltpu.VMEM((WB,), jnp.int32),
            pltpu.VMEM((WB,), jnp.int32),
            pltpu.VMEM((16,), jnp.int32),
            pltpu.SemaphoreType.DMA,
        ],
    )(z, wl, cnt)


def _dinv_body(hist_ref, o_ref):
    deg = hist_ref[...] + 1.0
    o_ref[...] = lax.rsqrt(deg)


def _mm_scale_body(x_ref, w_ref, dinv_ref, o_ref):
    o_ref[...] = jnp.dot(x_ref[...], w_ref[...],
                         preferred_element_type=jnp.float32) * dinv_ref[...]


def _layer2_body(acc_ref, dinv_ref, b_ref, w_ref, o_ref):
    h = jnp.maximum(acc_ref[...] * dinv_ref[...] + b_ref[...], 0.0)
    o_ref[...] = jnp.dot(h, w_ref[...],
                         preferred_element_type=jnp.float32) * dinv_ref[...]


def _final_body(acc_ref, dinv_ref, b_ref, o_ref):
    o_ref[...] = acc_ref[...] * dinv_ref[...] + b_ref[...]


def kernel(x, edge_index, W1, b1, W2, b2):
    n, d = x.shape
    e = edge_index.shape[1]
    assert n == N_NODES and d == D
    epad = -(-e // SCAN) * SCAN

    ei = edge_index.astype(jnp.int32)
    src_p = jnp.concatenate([ei[0], jnp.zeros((epad - e,), jnp.int32)])
    dst_p = jnp.concatenate(
        [ei[1], jnp.full((epad - e,), NPAD, jnp.int32)])
    x_p = jnp.concatenate(
        [x, jnp.zeros((NPAD - n, d), jnp.float32)], axis=0)
    b1r = b1.reshape(1, d)
    b2r = b2.reshape(1, d)

    wl, cnt, hist = _build_call(src_p, dst_p, epad)
    wl_cap = epad + SCAN + 512

    dinv_row = pl.pallas_call(
        _dinv_body,
        in_specs=[pl.BlockSpec((1, NPAD), lambda: (0, 0))],
        out_specs=pl.BlockSpec((1, NPAD), lambda: (0, 0)),
        out_shape=jax.ShapeDtypeStruct((1, NPAD), jnp.float32),
    )(hist.reshape(1, NPAD))
    dinv_col = dinv_row.reshape(NPAD, 1)

    grid = (NPAD // BM,)
    z1 = pl.pallas_call(
        _mm_scale_body,
        grid=grid,
        in_specs=[pl.BlockSpec((BM, d), lambda i: (i, 0)),
                  pl.BlockSpec((d, d), lambda i: (0, 0)),
                  pl.BlockSpec((BM, 1), lambda i: (i, 0))],
        out_specs=pl.BlockSpec((BM, d), lambda i: (i, 0)),
        out_shape=jax.ShapeDtypeStruct((NPAD, d), jnp.float32),
    )(x_p, W1, dinv_col)

    acc1 = _propagate_call(z1, wl, cnt, wl_cap)

    z2 = pl.pallas_call(
        _layer2_body,
        grid=grid,
        in_specs=[pl.BlockSpec((BM, d), lambda i: (i, 0)),
                  pl.BlockSpec((BM, 1), lambda i: (i, 0)),
                  pl.BlockSpec((1, d), lambda i: (0, 0)),
                  pl.BlockSpec((d, d), lambda i: (0, 0))],
        out_specs=pl.BlockSpec((BM, d), lambda i: (i, 0)),
        out_shape=jax.ShapeDtypeStruct((NPAD, d), jnp.float32),
    )(acc1, dinv_col, b1r, W2)

    acc2 = _propagate_call(z2, wl, cnt, wl_cap)

    out = pl.pallas_call(
        _final_body,
        grid=grid,
        in_specs=[pl.BlockSpec((BM, d), lambda i: (i, 0)),
                  pl.BlockSpec((BM, 1), lambda i: (i, 0)),
                  pl.BlockSpec((1, d), lambda i: (0, 0))],
        out_specs=pl.BlockSpec((BM, d), lambda i: (i, 0)),
        out_shape=jax.ShapeDtypeStruct((NPAD, d), jnp.float32),
    )(acc2, dinv_col, b2r)
    return out[:n]


# speedup vs baseline: 1.9192x; 1.3200x over previous
import functools

import jax
import jax.numpy as jnp
from jax import lax
from jax.experimental import pallas as pl
from jax.experimental.pallas import tpu as pltpu
from jax.experimental.pallas import tpu_sc as plsc

N_NODES = 10000
D = 256
NC = 2
NS = 16
NW = NC * NS

NPAD = 10240
RPT = NPAD // NW
SCAN = 8192
CHD = 64
WB = 2048
BM = 512
DUMMY_PACKED = RPT


def _sc_mesh():
    return plsc.VectorSubcoreMesh(
        core_axis_name="c", subcore_axis_name="s",
        num_cores=NC, num_subcores=NS)


def _cp():
    return pltpu.CompilerParams(needs_layout_passes=False)


def _build_body(epad, wl_cap, src_hbm, dst_hbm, wl_hbm, cnt_hbm, hist_hbm,
                srcc, dstc, wlb, histloc, cntbuf):
    cid = lax.axis_index("c")
    sid = lax.axis_index("s")
    wid = cid * NS + sid
    row_lo = wid * RPT

    zeros = jnp.zeros((16,), jnp.float32)

    def zb(i, c):
        histloc[pl.ds(i * 16, 16)] = zeros
        return c
    lax.fori_loop(0, RPT // 16, zb, 0)

    ones = jnp.ones((16,), jnp.float32)
    dummy16 = jnp.full((16,), DUMMY_PACKED, jnp.int32)
    iota16 = lax.iota(jnp.int32, 16)

    def sbody(sc_i, carry):
        g_off = carry
        off = sc_i * SCAN
        pltpu.sync_copy(src_hbm.at[pl.ds(off, SCAN)], srcc)
        pltpu.sync_copy(dst_hbm.at[pl.ds(off, SCAN)], dstc)

        def vbody(v, wc):
            d = dstc[pl.ds(v * 16, 16)]
            s = srcc[pl.ds(v * 16, 16)]
            loc = d - row_lo
            mask = loc.astype(jnp.uint32) < jnp.uint32(RPT)
            packed = (s << 9) | loc
            plsc.store_compressed(wlb.at[pl.ds(wc, 16)], packed, mask=mask)
            plsc.addupdate_scatter(histloc, [loc], ones, mask=mask)
            cnt = plsc.all_reduce_population_count(mask)
            return wc + cnt[0]
        wc = lax.fori_loop(0, SCAN // 16, vbody, 0)
        plsc.store_compressed(wlb.at[pl.ds(wc, 16)], dummy16,
                              mask=(iota16 >= 0))
        wc16 = ((wc + 15) // 16) * 16
        pltpu.sync_copy(
            wlb.at[pl.ds(0, SCAN)],
            wl_hbm.at[pl.ds(pl.multiple_of(wid * wl_cap + g_off, 16), SCAN)])
        return g_off + wc16
    g_off = lax.fori_loop(0, epad // SCAN, sbody, 0)

    for k in range(CHD // 16):
        wlb[pl.ds(k * 16, 16)] = dummy16
    pltpu.sync_copy(
        wlb.at[pl.ds(0, CHD)],
        wl_hbm.at[pl.ds(pl.multiple_of(wid * wl_cap + g_off, 16), CHD)])
    total = ((g_off + CHD - 1) // CHD) * CHD

    cntbuf[...] = jnp.broadcast_to(total, (16,)).astype(jnp.int32)
    pltpu.sync_copy(cntbuf, cnt_hbm.at[pl.ds(wid * 16, 16)])
    pltpu.sync_copy(histloc.at[pl.ds(0, RPT)],
                    hist_hbm.at[pl.ds(wid * RPT, RPT)])


def _propagate_body(wl_cap, z_hbm, wl_hbm, cnt_hbm, acc_hbm,
                    acc, gbuf, pbuf, srcb, locb, cntbuf, gsem):
    cid = lax.axis_index("c")
    sid = lax.axis_index("s")
    wid = cid * NS + sid
    row_lo = wid * RPT

    pltpu.sync_copy(z_hbm.at[pl.ds(row_lo, RPT)], acc.at[pl.ds(0, RPT)])

    pltpu.sync_copy(cnt_hbm.at[pl.ds(wid * 16, 16)], cntbuf)
    total = cntbuf[...][0]

    iota16 = lax.iota(jnp.int32, 16)
    splats = [jnp.full((16,), kl, jnp.int32) for kl in range(16)]
    colvs = [iota16 + kc * 16 for kc in range(D // 16)]

    def accumulate_batch(boff, slot):
        gslot = gbuf.at[slot]

        @functools.partial(plsc.parallel_loop, 0, CHD, unroll=8)
        def _(i):
            base = pl.multiple_of(boff + (i // 16) * 16, 16)
            locv = locb[pl.ds(base, 16)]
            lane = lax.rem(i, 16)
            rowsplat = locv[jnp.broadcast_to(lane, (16,))]
            for kc in range(D // 16):
                v = gslot[i, pl.ds(kc * 16, 16)]
                plsc.addupdate_scatter(acc, [rowsplat, colvs[kc]], v)

    nblocks = (total + WB - 1) // WB

    def bbody(bb, c):
        pltpu.sync_copy(
            wl_hbm.at[pl.ds(pl.multiple_of(wid * wl_cap + bb * WB, 16), WB)],
            pbuf)

        def ub(j, c2):
            v = pbuf[pl.ds(j * 16, 16)]
            srcb[pl.ds(j * 16, 16)] = v >> 9
            locb[pl.ds(j * 16, 16)] = v & 511
            return c2
        lax.fori_loop(0, WB // 16, ub, 0)

        rem = total - bb * WB
        nbb = jnp.minimum(rem, WB) // CHD

        @pl.when(nbb > 0)
        def _():
            pltpu.async_copy(
                z_hbm.at[srcb.at[pl.ds(0, CHD)]], gbuf.at[0], gsem)

            def dbody(b, c3):
                slot = lax.rem(b, 2)
                pltpu.make_async_copy(
                    z_hbm.at[srcb.at[pl.ds(0, CHD)]], gbuf.at[slot], gsem
                ).wait()

                @pl.when(b + 1 < nbb)
                def _():
                    pltpu.async_copy(
                        z_hbm.at[srcb.at[pl.ds((b + 1) * CHD, CHD)]],
                        gbuf.at[1 - slot], gsem)
                accumulate_batch(b * CHD, slot)
                return c3
            lax.fori_loop(0, nbb, dbody, 0)
        return c
    lax.fori_loop(0, nblocks, bbody, 0)

    pltpu.sync_copy(acc.at[pl.ds(0, RPT)],
                    acc_hbm.at[pl.ds(row_lo, RPT)])


def _build_call(src_p, dst_p, epad):
    wl_cap = epad + SCAN + 512
    return pl.kernel(
        functools.partial(_build_body, epad, wl_cap),
        out_type=(jax.ShapeDtypeStruct((NW * wl_cap,), jnp.int32),
                  jax.ShapeDtypeStruct((NW * 16,), jnp.int32),
                  jax.ShapeDtypeStruct((NW * RPT,), jnp.float32)),
        mesh=_sc_mesh(),
        compiler_params=_cp(),
        scratch_types=[
            pltpu.VMEM((SCAN,), jnp.int32),
            pltpu.VMEM((SCAN,), jnp.int32),
            pltpu.VMEM((SCAN + 32,), jnp.int32),
            pltpu.VMEM((RPT,), jnp.float32),
            pltpu.VMEM((16,), jnp.int32),
        ],
    )(src_p, dst_p)


def _propagate_call(z, wl, cnt, wl_cap):
    return pl.kernel(
        functools.partial(_propagate_body, wl_cap),
        out_type=jax.ShapeDtypeStruct((NPAD, D), jnp.float32),
        mesh=_sc_mesh(),
        compiler_params=_cp(),
        scratch_types=[
            pltpu.VMEM((RPT + 1, D), jnp.float32),
            pltpu.VMEM((2, CHD, D), jnp.float32),
            pltpu.VMEM((WB,), jnp.int32),
            pltpu.VMEM((WB,), jnp.int32),
            pltpu.VMEM((WB,), jnp.int32),
            pltpu.VMEM((16,), jnp.int32),
            pltpu.SemaphoreType.DMA,
        ],
    )(z, wl, cnt)


def _mm_scale_body(x_ref, w_ref, hist_ref, o_ref):
    dinv = lax.rsqrt(hist_ref[...] + 1.0)
    o_ref[...] = jnp.dot(x_ref[...], w_ref[...],
                         preferred_element_type=jnp.float32) * dinv


def _layer2_body(acc_ref, hist_ref, b_ref, w_ref, o_ref):
    dinv = lax.rsqrt(hist_ref[...] + 1.0)
    h = jnp.maximum(acc_ref[...] * dinv + b_ref[...], 0.0)
    o_ref[...] = jnp.dot(h, w_ref[...],
                         preferred_element_type=jnp.float32) * dinv


def _final_body(acc_ref, hist_ref, b_ref, o_ref):
    dinv = lax.rsqrt(hist_ref[...] + 1.0)
    o_ref[...] = acc_ref[...] * dinv + b_ref[...]


def kernel(x, edge_index, W1, b1, W2, b2):
    n, d = x.shape
    e = edge_index.shape[1]
    assert n == N_NODES and d == D
    epad = -(-e // SCAN) * SCAN

    ei = edge_index.astype(jnp.int32)
    src_p = jnp.concatenate([ei[0], jnp.zeros((epad - e,), jnp.int32)])
    dst_p = jnp.concatenate(
        [ei[1], jnp.full((epad - e,), NPAD, jnp.int32)])
    x_p = jnp.concatenate(
        [x, jnp.zeros((NPAD - n, d), jnp.float32)], axis=0)
    b1r = b1.reshape(1, d)
    b2r = b2.reshape(1, d)

    wl, cnt, hist = _build_call(src_p, dst_p, epad)
    wl_cap = epad + SCAN + 512

    hist_col = hist.reshape(NPAD, 1)

    grid = (NPAD // BM,)
    z1 = pl.pallas_call(
        _mm_scale_body,
        grid=grid,
        in_specs=[pl.BlockSpec((BM, d), lambda i: (i, 0)),
                  pl.BlockSpec((d, d), lambda i: (0, 0)),
                  pl.BlockSpec((BM, 1), lambda i: (i, 0))],
        out_specs=pl.BlockSpec((BM, d), lambda i: (i, 0)),
        out_shape=jax.ShapeDtypeStruct((NPAD, d), jnp.float32),
    )(x_p, W1, hist_col)

    acc1 = _propagate_call(z1, wl, cnt, wl_cap)

    z2 = pl.pallas_call(
        _layer2_body,
        grid=grid,
        in_specs=[pl.BlockSpec((BM, d), lambda i: (i, 0)),
                  pl.BlockSpec((BM, 1), lambda i: (i, 0)),
                  pl.BlockSpec((1, d), lambda i: (0, 0)),
                  pl.BlockSpec((d, d), lambda i: (0, 0))],
        out_specs=pl.BlockSpec((BM, d), lambda i: (i, 0)),
        out_shape=jax.ShapeDtypeStruct((NPAD, d), jnp.float32),
    )(acc1, hist_col, b1r, W2)

    acc2 = _propagate_call(z2, wl, cnt, wl_cap)

    out = pl.pallas_call(
        _final_body,
        grid=grid,
        in_specs=[pl.BlockSpec((BM, d), lambda i: (i, 0)),
                  pl.BlockSpec((BM, 1), lambda i: (i, 0)),
                  pl.BlockSpec((1, d), lambda i: (0, 0))],
        out_specs=pl.BlockSpec((BM, d), lambda i: (i, 0)),
        out_shape=jax.ShapeDtypeStruct((NPAD, d), jnp.float32),
    )(acc2, hist_col, b2r)
    return out[:n]
